# baseline (speedup 1.0000x reference)
import functools

import jax
import jax.numpy as jnp
from jax import lax
from jax.experimental import pallas as pl
from jax.experimental.pallas import tpu as pltpu
from jax.experimental.pallas import tpu_sc as plsc

EMBED_DIM = 16
PACK = 8
NUM_SC_CORES = 2
NUM_SC_SUBCORES = 16
NUM_WORKERS = NUM_SC_CORES * NUM_SC_SUBCORES
NBUF = 4
CHUNK = 256


def _repack_body(tbl3, out_hbm, n_rows, in_v, out_v, in_sems, out_sems,
                 wid, iota16):
    n_groups = n_rows // 128
    tail = n_rows - n_groups * 128
    gpw = (n_groups // NUM_WORKERS) // NBUF * NBUF
    extra = n_groups - gpw * NUM_WORKERS
    base_g = wid * gpw

    def fire_in(b, g):
        pltpu.async_copy(tbl3.at[0, :, pl.ds(g * 128, 128)],
                         in_v.at[b, 0:8, :], in_sems[b])
        pltpu.async_copy(tbl3.at[1, :, pl.ds(g * 128, 128)],
                         in_v.at[b, 8:16, :], in_sems[b])

    def shuffle(b, cols, lane_lo=0):
        for q in range(cols // 8):
            for k in range(8):
                lane = jnp.full((16,), lane_lo + 8 * q + k, jnp.int32)
                out_v[b, q, pl.ds(16 * k, 16)] = plsc.load_gather(
                    in_v.at[b], [iota16, lane])

    for b in range(NBUF):
        fire_in(b, base_g + b)

    @pl.loop(0, gpw, step=NBUF)
    def _(i0):
        for b in range(NBUF):
            g = base_g + i0 + b
            pltpu.make_async_copy(out_hbm.at[pl.ds(0, 16), :],
                                  in_v.at[b], in_sems[b]).wait()

            @pl.when(i0 > 0)
            def _():
                pltpu.make_async_copy(out_hbm.at[pl.ds(0, 16), :],
                                      out_v.at[b], out_sems[b]).wait()

            shuffle(b, 128)
            pltpu.async_copy(out_v.at[b], out_hbm.at[pl.ds(16 * g, 16), :],
                             out_sems[b])
            g_next = g + NBUF

            @pl.when(g_next < base_g + gpw)
            def _():
                fire_in(b, g_next)

    for b in range(NBUF):
        pltpu.make_async_copy(out_hbm.at[pl.ds(0, 16), :],
                              out_v.at[b], out_sems[b]).wait()

    @pl.when(wid < extra)
    def _():
        g = gpw * NUM_WORKERS + wid
        pltpu.sync_copy(tbl3.at[0, :, pl.ds(g * 128, 128)], in_v.at[0, 0:8, :])
        pltpu.sync_copy(tbl3.at[1, :, pl.ds(g * 128, 128)], in_v.at[0, 8:16, :])
        shuffle(0, 128)
        pltpu.sync_copy(out_v.at[0], out_hbm.at[pl.ds(16 * g, 16), :])

    if tail:
        @pl.when(wid == extra)
        def _():
            start = pl.multiple_of(n_groups * 128 + wid * 0, 128)
            pltpu.sync_copy(tbl3.at[0, :, pl.ds(start, 128)],
                            in_v.at[0, 0:8, :])
            pltpu.sync_copy(tbl3.at[1, :, pl.ds(start, 128)],
                            in_v.at[0, 8:16, :])
            shuffle(0, tail)
            pltpu.sync_copy(out_v.at[0, pl.ds(0, tail // 8), :],
                            out_hbm.at[pl.ds(16 * n_groups, tail // 8), :])


def _sc_repack(mt3, ut3):
    n_m = mt3.shape[2]
    n_u = ut3.shape[2]
    mesh = plsc.VectorSubcoreMesh(core_axis_name="c", subcore_axis_name="s")

    @functools.partial(
        pl.kernel,
        mesh=mesh,
        out_type=(
            jax.ShapeDtypeStruct((n_m // PACK, 128), jnp.float32),
            jax.ShapeDtypeStruct((n_u // PACK, 128), jnp.float32),
        ),
        scratch_types=(
            [pltpu.VMEM((NBUF, 16, 128), jnp.float32),
             pltpu.VMEM((NBUF, 16, 128), jnp.float32)]
            + [pltpu.SemaphoreType.DMA] * (2 * NBUF)
        ),
        compiler_params=pltpu.CompilerParams(use_tc_tiling_on_sc=True,
                                             needs_layout_passes=False),
    )
    def repack_kernel(mt_hbm, ut_hbm, mp_hbm, up_hbm, in_v, out_v, *sems):
        in_sems = sems[:NBUF]
        out_sems = sems[NBUF:]
        wid = lax.axis_index("s") * NUM_SC_CORES + lax.axis_index("c")
        iota16 = jax.lax.iota(jnp.int32, 16)
        _repack_body(ut_hbm, up_hbm, n_u, in_v, out_v, in_sems, out_sems,
                     wid, iota16)
        _repack_body(mt_hbm, mp_hbm, n_m, in_v, out_v, in_sems, out_sems,
                     wid, iota16)

    return repack_kernel(mt3, ut3)


def _sc_gather(mrow, urow, movie_packed, user_packed):
    batch = mrow.shape[0]
    b_per_w = batch // NUM_WORKERS
    mesh = plsc.VectorSubcoreMesh(core_axis_name="c", subcore_axis_name="s")

    @functools.partial(
        pl.kernel,
        mesh=mesh,
        out_type=(
            jax.ShapeDtypeStruct((batch, 128), jnp.float32),
            jax.ShapeDtypeStruct((batch, 128), jnp.float32),
        ),
        scratch_types=[
            pltpu.VMEM((b_per_w,), jnp.int32),
            pltpu.VMEM((b_per_w,), jnp.int32),
            pltpu.VMEM((CHUNK, 128), jnp.float32),
            pltpu.VMEM((CHUNK, 128), jnp.float32),
            pltpu.SemaphoreType.DMA,
            pltpu.SemaphoreType.DMA,
        ],
        compiler_params=pltpu.CompilerParams(use_tc_tiling_on_sc=True),
    )
    def gather_kernel(mt_hbm, ut_hbm, mi_hbm, ui_hbm, mo_hbm, uo_hbm,
                      mi_v, ui_v, mrows_v, urows_v, sem_m, sem_u):
        wid = lax.axis_index("s") * NUM_SC_CORES + lax.axis_index("c")
        base = wid * b_per_w
        pltpu.sync_copy(mi_hbm.at[pl.ds(base, b_per_w)], mi_v)
        pltpu.sync_copy(ui_hbm.at[pl.ds(base, b_per_w)], ui_v)

        @pl.loop(0, b_per_w, step=CHUNK)
        def _(c):
            cm = pltpu.async_copy(mt_hbm.at[mi_v.at[pl.ds(c, CHUNK)]],
                                  mrows_v, sem_m)
            cu = pltpu.async_copy(ut_hbm.at[ui_v.at[pl.ds(c, CHUNK)]],
                                  urows_v, sem_u)
            cm.wait()
            cu.wait()
            pltpu.sync_copy(mrows_v, mo_hbm.at[pl.ds(base + c, CHUNK)])
            pltpu.sync_copy(urows_v, uo_hbm.at[pl.ds(base + c, CHUNK)])

    return gather_kernel(movie_packed, user_packed, mrow, urow)


def _mlp_body(mc_ref, uc_ref, msub_ref, usub_ref, w0m_ref, w0u_ref, b0_ref,
              w1_ref, b1_ref, wo_ref, bo_ref, o_ref):
    col_group = jax.lax.broadcasted_iota(jnp.int32, (1, 128), 1) // EMBED_DIM
    mm = jnp.where(msub_ref[...] == col_group, mc_ref[...], 0.0)
    uu = jnp.where(usub_ref[...] == col_group, uc_ref[...], 0.0)
    h = (jnp.dot(mm, w0m_ref[...], preferred_element_type=jnp.float32)
         + jnp.dot(uu, w0u_ref[...], preferred_element_type=jnp.float32)
         + b0_ref[...])
    h = jnp.maximum(h, 0.0)
    h = jnp.dot(h, w1_ref[...], preferred_element_type=jnp.float32) + b1_ref[...]
    h = jnp.maximum(h, 0.0)
    o = jnp.dot(h, wo_ref[...], preferred_element_type=jnp.float32) + bo_ref[...]
    o_ref[...] = jax.nn.sigmoid(o)


def kernel(movie_id, user_id, movie_table, user_table, W0, b0, W1, b1, Wo, bo):
    batch = movie_id.shape[0]
    movie_id = movie_id.astype(jnp.int32)
    user_id = user_id.astype(jnp.int32)
    mt3 = movie_table.T.reshape(2, 8, movie_table.shape[0])
    ut3 = user_table.T.reshape(2, 8, user_table.shape[0])
    mp, up = _sc_repack(mt3, ut3)
    mrow = movie_id >> 3
    urow = user_id >> 3
    msub = (movie_id & 7)[:, None]
    usub = (user_id & 7)[:, None]
    mc, uc = _sc_gather(mrow, urow, mp, up)
    w0m = jnp.tile(W0[:, :EMBED_DIM].T, (PACK, 1))
    w0u = jnp.tile(W0[:, EMBED_DIM:].T, (PACK, 1))
    out = pl.pallas_call(
        _mlp_body,
        out_shape=jax.ShapeDtypeStruct((batch, 1), jnp.float32),
    )(mc, uc, msub, usub, w0m, w0u, b0[None, :], W1.T, b1[None, :],
      Wo.T, bo[None, :])
    return out
